# baseline (speedup 1.0000x reference)
import functools

import jax
import jax.numpy as jnp
from jax import lax
from jax.experimental import pallas as pl
from jax.experimental.pallas import tpu as pltpu
from jax.experimental.pallas import tpu_sc as plsc

N = 4096
E = 65536
D = 1024
P = 256

NW = 32
ROWS = 16
PASSES = N // (NW * ROWS)
CHUNK = 8192
LANES = 16


NCHUNK = E // CHUNK


def _adj_body(src_h, dst_h, a_h, acc, srcb, dstb, sem0, sem1):
    nc = 2
    wid = lax.axis_index("s") * nc + lax.axis_index("c")
    sems = (sem0, sem1)

    def pass_body(p, _):
        base = (p * NW + wid) * ROWS

        @functools.partial(plsc.parallel_loop, 0, ROWS * N // LANES,
                           unroll=8)
        def _zero(j):
            acc[pl.ds(j * LANES, LANES)] = jnp.zeros((LANES,), jnp.float32)

        def start(cc):
            b = cc & 1
            hs = pltpu.async_copy(
                src_h.at[pl.ds(cc * CHUNK, CHUNK)], srcb.at[b], sems[b])
            hd = pltpu.async_copy(
                dst_h.at[pl.ds(cc * CHUNK, CHUNK)], dstb.at[b], sems[b])
            return hs, hd

        pending = start(0)
        for cc in range(NCHUNK):
            b = cc & 1
            hs, hd = pending
            hs.wait()
            hd.wait()
            if cc + 1 < NCHUNK:
                pending = start(cc + 1)

            @functools.partial(plsc.parallel_loop, 0, CHUNK // LANES,
                               unroll=8)
            def _scan(j, b=b):
                s = srcb[b, pl.ds(j * LANES, LANES)]
                d = dstb[b, pl.ds(j * LANES, LANES)]
                loc = d - base
                m = (loc >= 0) & (loc < ROWS)
                flat = jnp.where(m, loc * N + s, 0)
                val = jnp.where(m, jnp.float32(1.0), jnp.float32(0.0))
                plsc.addupdate_scatter(acc, [flat], val)

        pltpu.sync_copy(acc, a_h.at[pl.ds(base * N, ROWS * N)])
        return ()
    lax.fori_loop(0, PASSES, pass_body, ())


def _build_adj(src, dst):
    mesh = plsc.VectorSubcoreMesh(core_axis_name="c", subcore_axis_name="s")
    a = pl.kernel(
        _adj_body,
        out_type=jax.ShapeDtypeStruct((N * N,), jnp.float32),
        mesh=mesh,
        compiler_params=pltpu.CompilerParams(needs_layout_passes=False),
        scratch_types=[
            pltpu.VMEM((ROWS * N,), jnp.float32),
            pltpu.VMEM((2, CHUNK), jnp.int32),
            pltpu.VMEM((2, CHUNK), jnp.int32),
            pltpu.SemaphoreType.DMA,
            pltpu.SemaphoreType.DMA,
        ],
    )(src, dst)
    return a.reshape(N, N)


BM = 512
BK = 1024


def _main_body(a_ref, xk_ref, xi_ref, wm_ref, wr_ref, w1_ref, b1_ref,
               y1_ref, st_ref, acc_ref):
    k = pl.program_id(1)

    @pl.when(k == 0)
    def _():
        acc_ref[...] = jnp.zeros_like(acc_ref)

    acc_ref[...] += jnp.dot(a_ref[...], xk_ref[...],
                            preferred_element_type=jnp.float32)

    @pl.when(k == N // BK - 1)
    def _():
        xi = xi_ref[...]
        t = jnp.dot(acc_ref[...], wm_ref[...],
                    preferred_element_type=jnp.float32)
        t += jnp.dot(xi, wr_ref[...], preferred_element_type=jnp.float32)
        h = xi + jnp.maximum(t, 0.0)
        y1 = jnp.dot(h, w1_ref[...],
                     preferred_element_type=jnp.float32) + b1_ref[...]
        y1_ref[...] = y1
        s = jnp.concatenate([jnp.sum(y1, axis=0, keepdims=True),
                             jnp.sum(y1 * y1, axis=0, keepdims=True)], axis=0)
        i = pl.program_id(0)

        @pl.when(i == 0)
        def _():
            st_ref[...] = s

        @pl.when(i > 0)
        def _():
            st_ref[...] += s


def _main_stage(a, x, wm, wr, w1, b1):
    grid = (N // BM, N // BK)
    return pl.pallas_call(
        _main_body,
        grid=grid,
        in_specs=[
            pl.BlockSpec((BM, BK), lambda i, k: (i, k)),
            pl.BlockSpec((BK, D), lambda i, k: (k, 0)),
            pl.BlockSpec((BM, D), lambda i, k: (i, 0)),
            pl.BlockSpec((D, D), lambda i, k: (0, 0)),
            pl.BlockSpec((D, D), lambda i, k: (0, 0)),
            pl.BlockSpec((D, D), lambda i, k: (0, 0)),
            pl.BlockSpec((1, D), lambda i, k: (0, 0)),
        ],
        out_specs=[
            pl.BlockSpec((BM, D), lambda i, k: (i, 0)),
            pl.BlockSpec((2, D), lambda i, k: (0, 0)),
        ],
        out_shape=[
            jax.ShapeDtypeStruct((N, D), jnp.float32),
            jax.ShapeDtypeStruct((2, D), jnp.float32),
        ],
        scratch_shapes=[pltpu.VMEM((BM, D), jnp.float32)],
    )(a.astype(jnp.bfloat16), x.astype(jnp.bfloat16), x, wm, wr, w1, b1)


def _proj2_body(y1_ref, st_ref, g_ref, be_ref, w2_ref, b2_ref,
                y2_ref, st2_ref):
    st = st_ref[...]
    mean = st[0:1, :] * (1.0 / N)
    var = st[1:2, :] * (1.0 / N) - mean * mean
    z = g_ref[...] * (y1_ref[...] - mean) * lax.rsqrt(var + 1e-5) + be_ref[...]
    z = jnp.maximum(z, 0.0)
    y2 = jnp.dot(z, w2_ref[...], preferred_element_type=jnp.float32) + b2_ref[...]
    y2_ref[...] = y2
    s = jnp.concatenate([jnp.sum(y2, axis=0, keepdims=True),
                         jnp.sum(y2 * y2, axis=0, keepdims=True)], axis=0)
    i = pl.program_id(0)

    @pl.when(i == 0)
    def _():
        st2_ref[...] = s

    @pl.when(i > 0)
    def _():
        st2_ref[...] += s


def _proj2_stage(y1, st1, g1, be1, w2, b2):
    grid = (N // BM,)
    return pl.pallas_call(
        _proj2_body,
        grid=grid,
        in_specs=[
            pl.BlockSpec((BM, D), lambda i: (i, 0)),
            pl.BlockSpec((2, D), lambda i: (0, 0)),
            pl.BlockSpec((1, D), lambda i: (0, 0)),
            pl.BlockSpec((1, D), lambda i: (0, 0)),
            pl.BlockSpec((D, P), lambda i: (0, 0)),
            pl.BlockSpec((1, P), lambda i: (0, 0)),
        ],
        out_specs=[
            pl.BlockSpec((BM, P), lambda i: (i, 0)),
            pl.BlockSpec((2, P), lambda i: (0, 0)),
        ],
        out_shape=[
            jax.ShapeDtypeStruct((N, P), jnp.float32),
            jax.ShapeDtypeStruct((2, P), jnp.float32),
        ],
    )(y1, st1, g1, be1, w2, b2)


def _norm_body(y2_ref, st_ref, g_ref, be_ref, e_ref):
    st = st_ref[...]
    mean = st[0:1, :] * (1.0 / N)
    var = st[1:2, :] * (1.0 / N) - mean * mean
    z = g_ref[...] * (y2_ref[...] - mean) * lax.rsqrt(var + 1e-5) + be_ref[...]
    z = jnp.maximum(z, 0.0)
    nrm = jnp.sqrt(jnp.sum(z * z, axis=1, keepdims=True))
    e_ref[...] = z / jnp.maximum(nrm, 1e-12)


def _norm_stage(y2, st2, g2, be2):
    grid = (N // BM,)
    return pl.pallas_call(
        _norm_body,
        grid=grid,
        in_specs=[
            pl.BlockSpec((BM, P), lambda i: (i, 0)),
            pl.BlockSpec((2, P), lambda i: (0, 0)),
            pl.BlockSpec((1, P), lambda i: (0, 0)),
            pl.BlockSpec((1, P), lambda i: (0, 0)),
        ],
        out_specs=pl.BlockSpec((BM, P), lambda i: (i, 0)),
        out_shape=jax.ShapeDtypeStruct((N, P), jnp.float32),
    )(y2, st2, g2, be2)


def _aff_body(es_ref, et_ref, o_ref):
    o_ref[...] = lax.dot_general(
        es_ref[...], et_ref[...], (((1,), (1,)), ((), ())),
        preferred_element_type=jnp.float32)


def _aff_stage(es, et):
    grid = (N // BM, N // BM)
    return pl.pallas_call(
        _aff_body,
        grid=grid,
        in_specs=[
            pl.BlockSpec((BM, P), lambda i, j: (i, 0)),
            pl.BlockSpec((BM, P), lambda i, j: (j, 0)),
        ],
        out_specs=pl.BlockSpec((BM, BM), lambda i, j: (i, j)),
        out_shape=jax.ShapeDtypeStruct((N, N), jnp.float32),
    )(es, et)


def _graph_embed(a, x, W_msg, W_root, W1, b1, g1, be1, W2, b2, g2, be2):
    y1, st1 = _main_stage(a, x, W_msg, W_root, W1, b1.reshape(1, D))
    y2, st2 = _proj2_stage(y1, st1, g1.reshape(1, D), be1.reshape(1, D),
                           W2, b2.reshape(1, P))
    return _norm_stage(y2, st2, g2.reshape(1, P), be2.reshape(1, P))


def kernel(x_src, edge_index_src, x_tgt, edge_index_tgt,
           W_msg, W_root, W1, b1, g1, be1, W2, b2, g2, be2):
    src_s = edge_index_src[0].astype(jnp.int32)
    dst_s = edge_index_src[1].astype(jnp.int32)
    src_t = edge_index_tgt[0].astype(jnp.int32)
    dst_t = edge_index_tgt[1].astype(jnp.int32)

    a_s = _build_adj(src_s, dst_s)
    a_t = _build_adj(src_t, dst_t)

    mlp = (W1, b1, g1, be1, W2, b2, g2, be2)
    es = _graph_embed(a_s, x_src, W_msg, W_root, *mlp)
    et = _graph_embed(a_t, x_tgt, W_msg, W_root, *mlp)
    return _aff_stage(es, et)


# speedup vs baseline: 1.3877x; 1.3877x over previous
import functools

import jax
import jax.numpy as jnp
from jax import lax
from jax.experimental import pallas as pl
from jax.experimental.pallas import tpu as pltpu
from jax.experimental.pallas import tpu_sc as plsc

N = 4096
E = 65536
D = 1024
P = 256

NW = 32
ROWS = 16
PASSES = N // (NW * ROWS)
CHUNK = 8192
LANES = 16


NSUB = 16
ESLICE = E // NSUB
CAP = 768
PASS_ROWS = NW * ROWS
SHIFT = PASS_ROWS.bit_length() - 1


def _adj_body(src_h, dst_h, a_h, acc, srcb, dstb, binbuf, passbuf, shared):
    nc = 2
    sid = lax.axis_index("s")
    wid = sid * nc + lax.axis_index("c")

    def sent_body(j, _):
        binbuf[pl.ds(j * LANES, LANES)] = jnp.full((LANES,), -1, jnp.int32)
        return ()
    lax.fori_loop(0, PASSES * CAP // LANES, sent_body, ())

    pltpu.sync_copy(src_h.at[pl.ds(sid * ESLICE, ESLICE)], srcb)
    pltpu.sync_copy(dst_h.at[pl.ds(sid * ESLICE, ESLICE)], dstb)

    zero16 = jnp.zeros((LANES,), jnp.int32)

    def bin_body(j, cnts):
        s = srcb[pl.ds(j * LANES, LANES)]
        d = dstb[pl.ds(j * LANES, LANES)]
        code = d * N + s
        bn = lax.shift_right_logical(d, SHIFT)
        out = []
        for b in range(PASSES):
            m = bn == b
            mi = jnp.where(m, 1, 0)
            cs = plsc.cumsum(mi)
            pos = jnp.minimum(cnts[b] + cs - 1, CAP - 1)
            plsc.store_scatter(binbuf, [b * CAP + pos], code, mask=m)
            tot = plsc.all_reduce_population_count(m)
            out.append(cnts[b] + tot)
        return tuple(out)

    lax.fori_loop(0, ESLICE // LANES, bin_body, (zero16,) * PASSES)

    for b in range(PASSES):
        pltpu.sync_copy(binbuf.at[pl.ds(b * CAP, CAP)],
                        shared.at[b, pl.ds(sid * CAP, CAP)])
    plsc.subcore_barrier()

    def pass_body(p, _):
        base = (p * NW + wid) * ROWS
        pltpu.sync_copy(shared.at[p], passbuf)

        @functools.partial(plsc.parallel_loop, 0, ROWS * N // LANES,
                           unroll=8)
        def _zero(j):
            acc[pl.ds(j * LANES, LANES)] = jnp.zeros((LANES,), jnp.float32)

        @functools.partial(plsc.parallel_loop, 0, NSUB * CAP // LANES,
                           unroll=8)
        def _scan(j):
            code = passbuf[pl.ds(j * LANES, LANES)]
            loc = lax.shift_right_arithmetic(code, 12) - base
            m = (code >= 0) & (loc >= 0) & (loc < ROWS)
            flat = jnp.where(m, loc * N + (code & (N - 1)), 0)
            val = jnp.where(m, jnp.float32(1.0), jnp.float32(0.0))
            plsc.addupdate_scatter(acc, [flat], val)

        pltpu.sync_copy(acc, a_h.at[pl.ds(base * N, ROWS * N)])
        return ()
    lax.fori_loop(0, PASSES, pass_body, ())


def _build_adj(src, dst):
    mesh = plsc.VectorSubcoreMesh(core_axis_name="c", subcore_axis_name="s")
    a = pl.kernel(
        _adj_body,
        out_type=jax.ShapeDtypeStruct((N * N,), jnp.float32),
        mesh=mesh,
        compiler_params=pltpu.CompilerParams(needs_layout_passes=False),
        scratch_types=[
            pltpu.VMEM((ROWS * N,), jnp.float32),
            pltpu.VMEM((ESLICE,), jnp.int32),
            pltpu.VMEM((ESLICE,), jnp.int32),
            pltpu.VMEM((PASSES * CAP,), jnp.int32),
            pltpu.VMEM((NSUB * CAP,), jnp.int32),
            pltpu.VMEM_SHARED((PASSES, NSUB * CAP), jnp.int32),
        ],
    )(src, dst)
    return a.reshape(N, N)


BM = 512
BK = 1024


def _main_body(a_ref, xk_ref, xi_ref, wm_ref, wr_ref, w1_ref, b1_ref,
               y1_ref, st_ref, acc_ref):
    k = pl.program_id(1)

    @pl.when(k == 0)
    def _():
        acc_ref[...] = jnp.zeros_like(acc_ref)

    acc_ref[...] += jnp.dot(a_ref[...], xk_ref[...],
                            preferred_element_type=jnp.float32)

    @pl.when(k == N // BK - 1)
    def _():
        xi = xi_ref[...]
        t = jnp.dot(acc_ref[...], wm_ref[...],
                    preferred_element_type=jnp.float32)
        t += jnp.dot(xi, wr_ref[...], preferred_element_type=jnp.float32)
        h = xi + jnp.maximum(t, 0.0)
        y1 = jnp.dot(h, w1_ref[...],
                     preferred_element_type=jnp.float32) + b1_ref[...]
        y1_ref[...] = y1
        s = jnp.concatenate([jnp.sum(y1, axis=0, keepdims=True),
                             jnp.sum(y1 * y1, axis=0, keepdims=True)], axis=0)
        i = pl.program_id(0)

        @pl.when(i == 0)
        def _():
            st_ref[...] = s

        @pl.when(i > 0)
        def _():
            st_ref[...] += s


def _main_stage(a, x, wm, wr, w1, b1):
    grid = (N // BM, N // BK)
    return pl.pallas_call(
        _main_body,
        grid=grid,
        in_specs=[
            pl.BlockSpec((BM, BK), lambda i, k: (i, k)),
            pl.BlockSpec((BK, D), lambda i, k: (k, 0)),
            pl.BlockSpec((BM, D), lambda i, k: (i, 0)),
            pl.BlockSpec((D, D), lambda i, k: (0, 0)),
            pl.BlockSpec((D, D), lambda i, k: (0, 0)),
            pl.BlockSpec((D, D), lambda i, k: (0, 0)),
            pl.BlockSpec((1, D), lambda i, k: (0, 0)),
        ],
        out_specs=[
            pl.BlockSpec((BM, D), lambda i, k: (i, 0)),
            pl.BlockSpec((2, D), lambda i, k: (0, 0)),
        ],
        out_shape=[
            jax.ShapeDtypeStruct((N, D), jnp.float32),
            jax.ShapeDtypeStruct((2, D), jnp.float32),
        ],
        scratch_shapes=[pltpu.VMEM((BM, D), jnp.float32)],
    )(a, x, x, wm, wr, w1, b1)


def _proj2_body(y1_ref, st_ref, g_ref, be_ref, w2_ref, b2_ref,
                y2_ref, st2_ref):
    st = st_ref[...]
    mean = st[0:1, :] * (1.0 / N)
    var = st[1:2, :] * (1.0 / N) - mean * mean
    z = g_ref[...] * (y1_ref[...] - mean) * lax.rsqrt(var + 1e-5) + be_ref[...]
    z = jnp.maximum(z, 0.0)
    y2 = jnp.dot(z, w2_ref[...], preferred_element_type=jnp.float32) + b2_ref[...]
    y2_ref[...] = y2
    s = jnp.concatenate([jnp.sum(y2, axis=0, keepdims=True),
                         jnp.sum(y2 * y2, axis=0, keepdims=True)], axis=0)
    i = pl.program_id(0)

    @pl.when(i == 0)
    def _():
        st2_ref[...] = s

    @pl.when(i > 0)
    def _():
        st2_ref[...] += s


def _proj2_stage(y1, st1, g1, be1, w2, b2):
    grid = (N // BM,)
    return pl.pallas_call(
        _proj2_body,
        grid=grid,
        in_specs=[
            pl.BlockSpec((BM, D), lambda i: (i, 0)),
            pl.BlockSpec((2, D), lambda i: (0, 0)),
            pl.BlockSpec((1, D), lambda i: (0, 0)),
            pl.BlockSpec((1, D), lambda i: (0, 0)),
            pl.BlockSpec((D, P), lambda i: (0, 0)),
            pl.BlockSpec((1, P), lambda i: (0, 0)),
        ],
        out_specs=[
            pl.BlockSpec((BM, P), lambda i: (i, 0)),
            pl.BlockSpec((2, P), lambda i: (0, 0)),
        ],
        out_shape=[
            jax.ShapeDtypeStruct((N, P), jnp.float32),
            jax.ShapeDtypeStruct((2, P), jnp.float32),
        ],
    )(y1, st1, g1, be1, w2, b2)


def _norm_body(y2_ref, st_ref, g_ref, be_ref, e_ref):
    st = st_ref[...]
    mean = st[0:1, :] * (1.0 / N)
    var = st[1:2, :] * (1.0 / N) - mean * mean
    z = g_ref[...] * (y2_ref[...] - mean) * lax.rsqrt(var + 1e-5) + be_ref[...]
    z = jnp.maximum(z, 0.0)
    nrm = jnp.sqrt(jnp.sum(z * z, axis=1, keepdims=True))
    e_ref[...] = z / jnp.maximum(nrm, 1e-12)


def _norm_stage(y2, st2, g2, be2):
    grid = (N // BM,)
    return pl.pallas_call(
        _norm_body,
        grid=grid,
        in_specs=[
            pl.BlockSpec((BM, P), lambda i: (i, 0)),
            pl.BlockSpec((2, P), lambda i: (0, 0)),
            pl.BlockSpec((1, P), lambda i: (0, 0)),
            pl.BlockSpec((1, P), lambda i: (0, 0)),
        ],
        out_specs=pl.BlockSpec((BM, P), lambda i: (i, 0)),
        out_shape=jax.ShapeDtypeStruct((N, P), jnp.float32),
    )(y2, st2, g2, be2)


def _aff_body(es_ref, et_ref, o_ref):
    o_ref[...] = lax.dot_general(
        es_ref[...], et_ref[...], (((1,), (1,)), ((), ())),
        preferred_element_type=jnp.float32)


def _aff_stage(es, et):
    grid = (N // BM, N // BM)
    return pl.pallas_call(
        _aff_body,
        grid=grid,
        in_specs=[
            pl.BlockSpec((BM, P), lambda i, j: (i, 0)),
            pl.BlockSpec((BM, P), lambda i, j: (j, 0)),
        ],
        out_specs=pl.BlockSpec((BM, BM), lambda i, j: (i, j)),
        out_shape=jax.ShapeDtypeStruct((N, N), jnp.float32),
    )(es, et)


def _graph_embed(a, x, W_msg, W_root, W1, b1, g1, be1, W2, b2, g2, be2):
    y1, st1 = _main_stage(a, x, W_msg, W_root, W1, b1.reshape(1, D))
    y2, st2 = _proj2_stage(y1, st1, g1.reshape(1, D), be1.reshape(1, D),
                           W2, b2.reshape(1, P))
    return _norm_stage(y2, st2, g2.reshape(1, P), be2.reshape(1, P))


def kernel(x_src, edge_index_src, x_tgt, edge_index_tgt,
           W_msg, W_root, W1, b1, g1, be1, W2, b2, g2, be2):
    src_s = edge_index_src[0].astype(jnp.int32)
    dst_s = edge_index_src[1].astype(jnp.int32)
    src_t = edge_index_tgt[0].astype(jnp.int32)
    dst_t = edge_index_tgt[1].astype(jnp.int32)

    a_s = _build_adj(src_s, dst_s)
    a_t = _build_adj(src_t, dst_t)

    mlp = (W1, b1, g1, be1, W2, b2, g2, be2)
    es = _graph_embed(a_s, x_src, W_msg, W_root, *mlp)
    et = _graph_embed(a_t, x_tgt, W_msg, W_root, *mlp)
    return _aff_stage(es, et)


# speedup vs baseline: 1.5089x; 1.0873x over previous
import functools

import jax
import jax.numpy as jnp
from jax import lax
from jax.experimental import pallas as pl
from jax.experimental.pallas import tpu as pltpu
from jax.experimental.pallas import tpu_sc as plsc

N = 4096
E = 65536
D = 1024
P = 256

NW = 32
ROWS = 16
PASSES = N // (NW * ROWS)
CHUNK = 8192
LANES = 16


NSUB = 16
ESLICE = E // NSUB
CAP = 768
PASS_ROWS = NW * ROWS
SHIFT = PASS_ROWS.bit_length() - 1


def _adj_body(src_h, dst_h, a_h, acc, srcb, dstb, binbuf, passbuf, shared):
    nc = 2
    sid = lax.axis_index("s")
    wid = sid * nc + lax.axis_index("c")

    def sent_body(j, _):
        binbuf[pl.ds(j * LANES, LANES)] = jnp.full((LANES,), -1, jnp.int32)
        return ()
    lax.fori_loop(0, PASSES * CAP // LANES, sent_body, ())

    pltpu.sync_copy(src_h.at[pl.ds(sid * ESLICE, ESLICE)], srcb)
    pltpu.sync_copy(dst_h.at[pl.ds(sid * ESLICE, ESLICE)], dstb)

    zero16 = jnp.zeros((LANES,), jnp.int32)

    def bin_body(j, cnts):
        s = srcb[pl.ds(j * LANES, LANES)]
        d = dstb[pl.ds(j * LANES, LANES)]
        code = d * N + s
        bn = lax.shift_right_logical(d, SHIFT)
        out = []
        for b in range(PASSES):
            m = bn == b
            mi = jnp.where(m, 1, 0)
            cs = plsc.cumsum(mi)
            pos = jnp.minimum(cnts[b] + cs - 1, CAP - 1)
            plsc.store_scatter(binbuf, [b * CAP + pos], code, mask=m)
            tot = plsc.all_reduce_population_count(m)
            out.append(cnts[b] + tot)
        return tuple(out)

    lax.fori_loop(0, ESLICE // LANES, bin_body, (zero16,) * PASSES)

    for b in range(PASSES):
        pltpu.sync_copy(binbuf.at[pl.ds(b * CAP, CAP)],
                        shared.at[b, pl.ds(sid * CAP, CAP)])
    plsc.subcore_barrier()

    def pass_body(p, _):
        base = (p * NW + wid) * ROWS
        pltpu.sync_copy(shared.at[p], passbuf)

        @functools.partial(plsc.parallel_loop, 0, ROWS * N // LANES,
                           unroll=8)
        def _zero(j):
            acc[pl.ds(j * LANES, LANES)] = jnp.zeros((LANES,), jnp.float32)

        @functools.partial(plsc.parallel_loop, 0, NSUB * CAP // LANES,
                           unroll=8)
        def _scan(j):
            code = passbuf[pl.ds(j * LANES, LANES)]
            loc = lax.shift_right_arithmetic(code, 12) - base
            m = (code >= 0) & (loc >= 0) & (loc < ROWS)
            flat = jnp.where(m, loc * N + (code & (N - 1)), 0)
            val = jnp.where(m, jnp.float32(1.0), jnp.float32(0.0))
            plsc.addupdate_scatter(acc, [flat], val)

        pltpu.sync_copy(acc, a_h.at[pl.ds(base * N, ROWS * N)])
        return ()
    lax.fori_loop(0, PASSES, pass_body, ())


def _build_adj(src, dst):
    mesh = plsc.VectorSubcoreMesh(core_axis_name="c", subcore_axis_name="s")
    a = pl.kernel(
        _adj_body,
        out_type=jax.ShapeDtypeStruct((N * N,), jnp.float32),
        mesh=mesh,
        compiler_params=pltpu.CompilerParams(needs_layout_passes=False),
        scratch_types=[
            pltpu.VMEM((ROWS * N,), jnp.float32),
            pltpu.VMEM((ESLICE,), jnp.int32),
            pltpu.VMEM((ESLICE,), jnp.int32),
            pltpu.VMEM((PASSES * CAP,), jnp.int32),
            pltpu.VMEM((NSUB * CAP,), jnp.int32),
            pltpu.VMEM_SHARED((PASSES, NSUB * CAP), jnp.int32),
        ],
    )(src, dst)
    return a.reshape(N, N)


BM = 512
BK = 1024


def _main_body(a_ref, x_ref, wm_ref, wr_ref, w1_ref, b1_ref,
               y1_ref, st_ref, acc_ref):
    k = pl.program_id(1)

    @pl.when(k == 0)
    def _():
        acc_ref[...] = jnp.zeros_like(acc_ref)

    xk = x_ref[pl.ds(k * BK, BK), :]
    acc_ref[...] += jnp.dot(a_ref[...], xk,
                            preferred_element_type=jnp.float32)

    @pl.when(k == N // BK - 1)
    def _():
        i = pl.program_id(0)
        xi = x_ref[pl.ds(i * BM, BM), :]
        t = jnp.dot(acc_ref[...], wm_ref[...],
                    preferred_element_type=jnp.float32)
        t += jnp.dot(xi, wr_ref[...], preferred_element_type=jnp.float32)
        h = xi + jnp.maximum(t, 0.0)
        y1 = jnp.dot(h, w1_ref[...],
                     preferred_element_type=jnp.float32) + b1_ref[...]
        y1_ref[...] = y1
        s = jnp.concatenate([jnp.sum(y1, axis=0, keepdims=True),
                             jnp.sum(y1 * y1, axis=0, keepdims=True)], axis=0)

        @pl.when(i == 0)
        def _():
            st_ref[...] = s

        @pl.when(i > 0)
        def _():
            st_ref[...] += s


def _main_stage(a, x, wm, wr, w1, b1):
    grid = (N // BM, N // BK)
    return pl.pallas_call(
        _main_body,
        grid=grid,
        in_specs=[
            pl.BlockSpec((BM, BK), lambda i, k: (i, k)),
            pl.BlockSpec((N, D), lambda i, k: (0, 0)),
            pl.BlockSpec((D, D), lambda i, k: (0, 0)),
            pl.BlockSpec((D, D), lambda i, k: (0, 0)),
            pl.BlockSpec((D, D), lambda i, k: (0, 0)),
            pl.BlockSpec((1, D), lambda i, k: (0, 0)),
        ],
        out_specs=[
            pl.BlockSpec((BM, D), lambda i, k: (i, 0)),
            pl.BlockSpec((2, D), lambda i, k: (0, 0)),
        ],
        out_shape=[
            jax.ShapeDtypeStruct((N, D), jnp.float32),
            jax.ShapeDtypeStruct((2, D), jnp.float32),
        ],
        scratch_shapes=[pltpu.VMEM((BM, D), jnp.float32)],
    )(a, x, wm, wr, w1, b1)


def _proj2_body(y1_ref, st_ref, g_ref, be_ref, w2_ref, b2_ref,
                y2_ref, st2_ref):
    st = st_ref[...]
    mean = st[0:1, :] * (1.0 / N)
    var = st[1:2, :] * (1.0 / N) - mean * mean
    z = g_ref[...] * (y1_ref[...] - mean) * lax.rsqrt(var + 1e-5) + be_ref[...]
    z = jnp.maximum(z, 0.0)
    y2 = jnp.dot(z, w2_ref[...], preferred_element_type=jnp.float32) + b2_ref[...]
    y2_ref[...] = y2
    s = jnp.concatenate([jnp.sum(y2, axis=0, keepdims=True),
                         jnp.sum(y2 * y2, axis=0, keepdims=True)], axis=0)
    i = pl.program_id(0)

    @pl.when(i == 0)
    def _():
        st2_ref[...] = s

    @pl.when(i > 0)
    def _():
        st2_ref[...] += s


def _proj2_stage(y1, st1, g1, be1, w2, b2):
    grid = (N // BM,)
    return pl.pallas_call(
        _proj2_body,
        grid=grid,
        in_specs=[
            pl.BlockSpec((BM, D), lambda i: (i, 0)),
            pl.BlockSpec((2, D), lambda i: (0, 0)),
            pl.BlockSpec((1, D), lambda i: (0, 0)),
            pl.BlockSpec((1, D), lambda i: (0, 0)),
            pl.BlockSpec((D, P), lambda i: (0, 0)),
            pl.BlockSpec((1, P), lambda i: (0, 0)),
        ],
        out_specs=[
            pl.BlockSpec((BM, P), lambda i: (i, 0)),
            pl.BlockSpec((2, P), lambda i: (0, 0)),
        ],
        out_shape=[
            jax.ShapeDtypeStruct((N, P), jnp.float32),
            jax.ShapeDtypeStruct((2, P), jnp.float32),
        ],
    )(y1, st1, g1, be1, w2, b2)


def _bn_relu_norm(y2, st, g, be):
    mean = st[0:1, :] * (1.0 / N)
    var = st[1:2, :] * (1.0 / N) - mean * mean
    z = g * (y2 - mean) * lax.rsqrt(var + 1e-5) + be
    z = jnp.maximum(z, 0.0)
    nrm = jnp.sqrt(jnp.sum(z * z, axis=1, keepdims=True))
    return z / jnp.maximum(nrm, 1e-12)


def _normaff_body(y2s_ref, sts_ref, y2t_ref, stt_ref, g_ref, be_ref, o_ref):
    es = _bn_relu_norm(y2s_ref[...], sts_ref[...], g_ref[...], be_ref[...])
    et = _bn_relu_norm(y2t_ref[...], stt_ref[...], g_ref[...], be_ref[...])
    o_ref[...] = lax.dot_general(
        es, et, (((1,), (1,)), ((), ())),
        preferred_element_type=jnp.float32)


def _normaff_stage(y2s, sts, y2t, stt, g2, be2):
    grid = (N // BM, N // BM)
    return pl.pallas_call(
        _normaff_body,
        grid=grid,
        in_specs=[
            pl.BlockSpec((BM, P), lambda i, j: (i, 0)),
            pl.BlockSpec((2, P), lambda i, j: (0, 0)),
            pl.BlockSpec((BM, P), lambda i, j: (j, 0)),
            pl.BlockSpec((2, P), lambda i, j: (0, 0)),
            pl.BlockSpec((1, P), lambda i, j: (0, 0)),
            pl.BlockSpec((1, P), lambda i, j: (0, 0)),
        ],
        out_specs=pl.BlockSpec((BM, BM), lambda i, j: (i, j)),
        out_shape=jax.ShapeDtypeStruct((N, N), jnp.float32),
    )(y2s, sts, y2t, stt, g2, be2)


def _graph_embed(a, x, W_msg, W_root, W1, b1, g1, be1, W2, b2):
    y1, st1 = _main_stage(a, x, W_msg, W_root, W1, b1.reshape(1, D))
    return _proj2_stage(y1, st1, g1.reshape(1, D), be1.reshape(1, D),
                        W2, b2.reshape(1, P))


def kernel(x_src, edge_index_src, x_tgt, edge_index_tgt,
           W_msg, W_root, W1, b1, g1, be1, W2, b2, g2, be2):
    src_s = edge_index_src[0].astype(jnp.int32)
    dst_s = edge_index_src[1].astype(jnp.int32)
    src_t = edge_index_tgt[0].astype(jnp.int32)
    dst_t = edge_index_tgt[1].astype(jnp.int32)

    a_s = _build_adj(src_s, dst_s)
    a_t = _build_adj(src_t, dst_t)

    mlp = (W1, b1, g1, be1, W2, b2)
    y2s, sts = _graph_embed(a_s, x_src, W_msg, W_root, *mlp)
    y2t, stt = _graph_embed(a_t, x_tgt, W_msg, W_root, *mlp)
    return _normaff_stage(y2s, sts, y2t, stt,
                          g2.reshape(1, P), be2.reshape(1, P))
